# baseline (speedup 1.0000x reference)
import functools

import jax
import jax.numpy as jnp
from jax import lax
from jax.experimental import pallas as pl
from jax.experimental.pallas import tpu as pltpu
from jax.experimental.pallas import tpu_sc as plsc

V = 1000000
VH = V // 2
F = 26
D = 32
B = 4096

NC = 2
NS = 16
NW = NC * NS
N = B * F
NPW = N // NW
CH = 128
NCHUNK = NPW // CH


def _sc_gather(x_resh, emb_a, emb_b, lin_flat):
    mesh = plsc.VectorSubcoreMesh(
        core_axis_name="c", subcore_axis_name="s",
        num_cores=NC, num_subcores=NS)

    @functools.partial(
        pl.kernel,
        out_type=(
            jax.ShapeDtypeStruct((NW, NCHUNK, CH, D), jnp.float32),
            jax.ShapeDtypeStruct((NW, NCHUNK, CH), jnp.float32),
        ),
        mesh=mesh,
        scratch_types=[
            pltpu.VMEM((NCHUNK, CH), jnp.int32),
            pltpu.VMEM((2, CH), jnp.int32),
            pltpu.VMEM((2, CH), jnp.int32),
            pltpu.VMEM((2, CH, D), jnp.float32),
            pltpu.VMEM((2, CH, D), jnp.float32),
            pltpu.VMEM((2, CH, D), jnp.float32),
            pltpu.VMEM((NCHUNK, CH), jnp.float32),
            pltpu.SemaphoreType.DMA,
            pltpu.SemaphoreType.DMA,
            pltpu.SemaphoreType.DMA,
        ],
        compiler_params=pltpu.CompilerParams(use_tc_tiling_on_sc=False),
    )
    def k(x_hbm, ea_hbm, eb_hbm, lin_hbm, out_h, out_l,
          idx_v, ia_v, ib_v, a_v, b_v, comb_v, lin_v, sem_a, sem_b, sem_l):
        wid = lax.axis_index("s") * NC + lax.axis_index("c")
        pltpu.sync_copy(x_hbm.at[wid], idx_v)

        def prep(c, slot):
            for g in range(CH // 16):
                vals = idx_v[c, g * 16:(g + 1) * 16]
                ia_v[slot, g * 16:(g + 1) * 16] = jnp.minimum(vals, VH - 1)
                ib_v[slot, g * 16:(g + 1) * 16] = jnp.maximum(vals - VH, 0)

        def a_cpy(slot):
            return pltpu.make_async_copy(
                ea_hbm.at[ia_v.at[slot]], a_v.at[slot], sem_a)

        def b_cpy(slot):
            return pltpu.make_async_copy(
                eb_hbm.at[ib_v.at[slot]], b_v.at[slot], sem_b)

        def lin_cpy(c):
            return pltpu.make_async_copy(
                lin_hbm.at[idx_v.at[c]], lin_v.at[c], sem_l)

        def combine(c, slot):
            for g in range(CH // 16):
                vals = idx_v[c, g * 16:(g + 1) * 16]
                w = jnp.where(vals < VH, 1.0, 0.0)
                for r in range(16):
                    row = g * 16 + r
                    wr = w[r]
                    a0 = a_v[slot, row, 0:16]
                    b0 = b_v[slot, row, 0:16]
                    comb_v[slot, row, 0:16] = b0 + (a0 - b0) * wr
                    a1 = a_v[slot, row, 16:32]
                    b1 = b_v[slot, row, 16:32]
                    comb_v[slot, row, 16:32] = b1 + (a1 - b1) * wr

        prep(0, 0)
        a_cpy(0).start()
        b_cpy(0).start()
        lin_cpy(0).start()

        def body(c, _):
            slot = lax.rem(c, 2)
            nslot = lax.rem(c + 1, 2)
            prep(c + 1, nslot)
            a_cpy(nslot).start()
            b_cpy(nslot).start()
            lin_cpy(c + 1).start()
            a_cpy(slot).wait()
            b_cpy(slot).wait()
            lin_cpy(c).wait()
            combine(c, slot)
            pltpu.sync_copy(comb_v.at[slot], out_h.at[wid, c])
            return 0

        lax.fori_loop(0, NCHUNK - 1, body, 0)
        last = NCHUNK - 1
        lslot = last % 2
        a_cpy(lslot).wait()
        b_cpy(lslot).wait()
        lin_cpy(last).wait()
        combine(last, lslot)
        pltpu.sync_copy(comb_v.at[lslot], out_h.at[wid, last])
        pltpu.sync_copy(lin_v, out_l.at[wid])

    return k(x_resh, emb_a, emb_b, lin_flat)


def _tc_body(h_ref, lin_ref, sel_ref, w1, b1, w2, b2, w3, b3, w4, b4,
             o_ref):
    h = h_ref[...]
    sel = sel_ref[...]
    s = jnp.dot(h, sel, preferred_element_type=jnp.float32)
    sos = jnp.dot(h * h, sel, preferred_element_type=jnp.float32)
    ix = jnp.sum(s * s - sos, axis=1, keepdims=True)
    lin = jnp.sum(lin_ref[...], axis=1, keepdims=True)
    a = jnp.maximum(
        jnp.dot(h, w1[...], preferred_element_type=jnp.float32) + b1[...], 0.0)
    a = jnp.maximum(
        jnp.dot(a, w2[...], preferred_element_type=jnp.float32) + b2[...], 0.0)
    a = jnp.maximum(
        jnp.dot(a, w3[...], preferred_element_type=jnp.float32) + b3[...], 0.0)
    m = jnp.dot(a, w4[...], preferred_element_type=jnp.float32) + b4[...]
    o_ref[...] = jax.nn.sigmoid(lin + 0.5 * ix + m)


def _tc_fused(h, lin, sel, W1, b1, W2, b2, W3, b3, W4, b4):
    bs = 512
    grid = (B // bs,)
    H = F * D
    const = lambda shape: pl.BlockSpec(shape, lambda i: (0, 0))
    return pl.pallas_call(
        _tc_body,
        grid=grid,
        in_specs=[
            pl.BlockSpec((bs, H), lambda i: (i, 0)),
            pl.BlockSpec((bs, F), lambda i: (i, 0)),
            const((H, D)),
            const((H, 300)), const((1, 300)),
            const((300, 300)), const((1, 300)),
            const((300, 300)), const((1, 300)),
            const((300, 1)), const((1, 1)),
        ],
        out_specs=pl.BlockSpec((bs, 1), lambda i: (i, 0)),
        out_shape=jax.ShapeDtypeStruct((B, 1), jnp.float32),
    )(h, lin, sel, W1, b1, W2, b2, W3, b3, W4, b4)


def kernel(x, linear_table, emb_table, W1, b1, W2, b2, W3, b3, W4, b4):
    x_resh = x.astype(jnp.int32).reshape(NW, NCHUNK, CH)
    rows, lin_rows = _sc_gather(x_resh, emb_table[:VH], emb_table[VH:],
                                linear_table.reshape(V))
    h = rows.reshape(B, F * D)
    lin = lin_rows.reshape(B, F)
    sel = jnp.tile(jnp.eye(D, dtype=jnp.float32), (F, 1))
    return _tc_fused(h, lin, sel, W1,
                     b1.reshape(1, 300), W2, b2.reshape(1, 300),
                     W3, b3.reshape(1, 300), W4, b4.reshape(1, 1))


# speedup vs baseline: 2.3866x; 2.3866x over previous
import functools

import jax
import jax.numpy as jnp
from jax import lax
from jax.experimental import pallas as pl
from jax.experimental.pallas import tpu as pltpu
from jax.experimental.pallas import tpu_sc as plsc

V = 1000000
F = 26
D = 32
B = 4096

NC = 2
NS = 16
NW = NC * NS
N = B * F
NPW = N // NW
CH = 128
NCHUNK = NPW // CH


def _sc_gather(x_resh, emb_table, lin_flat):
    mesh = plsc.VectorSubcoreMesh(
        core_axis_name="c", subcore_axis_name="s",
        num_cores=NC, num_subcores=NS)

    @functools.partial(
        pl.kernel,
        out_type=(
            jax.ShapeDtypeStruct((NW, NCHUNK, CH, D), jnp.float32),
            jax.ShapeDtypeStruct((NW, NCHUNK, CH), jnp.float32),
        ),
        mesh=mesh,
        scratch_types=[
            pltpu.VMEM((NCHUNK, CH), jnp.int32),
            pltpu.VMEM((NCHUNK, CH, D), jnp.float32),
            pltpu.VMEM((NCHUNK, CH), jnp.float32),
            pltpu.SemaphoreType.DMA,
            pltpu.SemaphoreType.DMA,
        ],
        compiler_params=pltpu.CompilerParams(use_tc_tiling_on_sc=False),
    )
    def k(x_hbm, emb_hbm, lin_hbm, out_h, out_l,
          idx_v, rows_v, lin_v, sem_e, sem_l):
        wid = lax.axis_index("s") * NC + lax.axis_index("c")
        pltpu.sync_copy(x_hbm.at[wid], idx_v)

        def emb_cpy(c):
            return pltpu.make_async_copy(
                emb_hbm.at[idx_v.at[c]], rows_v.at[c], sem_e)

        def lin_cpy(c):
            return pltpu.make_async_copy(
                lin_hbm.at[idx_v.at[c]], lin_v.at[c], sem_l)

        emb_cpy(0).start()
        lin_cpy(0).start()

        def body(c, _):
            emb_cpy(c + 1).start()
            lin_cpy(c + 1).start()
            emb_cpy(c).wait()
            lin_cpy(c).wait()
            return 0

        lax.fori_loop(0, NCHUNK - 1, body, 0)
        emb_cpy(NCHUNK - 1).wait()
        lin_cpy(NCHUNK - 1).wait()

        pltpu.sync_copy(rows_v, out_h.at[wid])
        pltpu.sync_copy(lin_v, out_l.at[wid])

    return k(x_resh, emb_table, lin_flat)


def _tc_body(h_ref, lin_ref, sel_ref, w1, b1, w2, b2, w3, b3, w4, b4,
             o_ref):
    h = h_ref[...]
    sel = sel_ref[...]
    s = jnp.dot(h, sel, preferred_element_type=jnp.float32)
    sos = jnp.dot(h * h, sel, preferred_element_type=jnp.float32)
    ix = jnp.sum(s * s - sos, axis=1, keepdims=True)
    lin = jnp.sum(lin_ref[...], axis=1, keepdims=True)
    a = jnp.maximum(
        jnp.dot(h, w1[...], preferred_element_type=jnp.float32) + b1[...], 0.0)
    a = jnp.maximum(
        jnp.dot(a, w2[...], preferred_element_type=jnp.float32) + b2[...], 0.0)
    a = jnp.maximum(
        jnp.dot(a, w3[...], preferred_element_type=jnp.float32) + b3[...], 0.0)
    m = jnp.dot(a, w4[...], preferred_element_type=jnp.float32) + b4[...]
    o_ref[...] = jax.nn.sigmoid(lin + 0.5 * ix + m)


def _tc_fused(h, lin, sel, W1, b1, W2, b2, W3, b3, W4, b4):
    bs = 512
    grid = (B // bs,)
    H = F * D
    const = lambda shape: pl.BlockSpec(shape, lambda i: (0, 0))
    return pl.pallas_call(
        _tc_body,
        grid=grid,
        in_specs=[
            pl.BlockSpec((bs, H), lambda i: (i, 0)),
            pl.BlockSpec((bs, F), lambda i: (i, 0)),
            const((H, D)),
            const((H, 300)), const((1, 300)),
            const((300, 300)), const((1, 300)),
            const((300, 300)), const((1, 300)),
            const((300, 1)), const((1, 1)),
        ],
        out_specs=pl.BlockSpec((bs, 1), lambda i: (i, 0)),
        out_shape=jax.ShapeDtypeStruct((B, 1), jnp.float32),
    )(h, lin, sel, W1, b1, W2, b2, W3, b3, W4, b4)


def kernel(x, linear_table, emb_table, W1, b1, W2, b2, W3, b3, W4, b4):
    x_resh = x.astype(jnp.int32).reshape(NW, NCHUNK, CH)
    rows, lin_rows = _sc_gather(x_resh, emb_table, linear_table.reshape(V))
    h = rows.reshape(B, F * D)
    lin = lin_rows.reshape(B, F)
    sel = jnp.tile(jnp.eye(D, dtype=jnp.float32), (F, 1))
    return _tc_fused(h, lin, sel, W1,
                     b1.reshape(1, 300), W2, b2.reshape(1, 300),
                     W3, b3.reshape(1, 300), W4, b4.reshape(1, 1))
